# Initial kernel scaffold; baseline (speedup 1.0000x reference)
import jax
import jax.numpy as jnp
from jax.experimental import pallas as pl


def kernel(x_nhwc, a, w2, bias):
    raise NotImplementedError("write your pallas kernel here")


# speedup vs baseline: 1.4083x; 1.4083x over previous
import jax
import jax.numpy as jnp
from jax.experimental import pallas as pl
from jax.experimental.pallas import tpu as pltpu

_P = 4
_PAD = 16


def _fused_kernel(x_ref, a_ref, w2_ref, b_ref, o_ref, s_ref):
    H = o_ref.shape[1]
    Kin = x_ref.shape[2]
    Kout = o_ref.shape[2]

    s_ref[0:_PAD] = jnp.zeros((_PAD, _P * Kin), s_ref.dtype)
    s_ref[H + _PAD:H + 2 * _PAD] = jnp.zeros((_PAD, _P * Kin), s_ref.dtype)
    for i in range(_P):
        s_ref[_PAD:H + _PAD, i * Kin:(i + 1) * Kin] = x_ref[i].astype(s_ref.dtype)

    bias = b_ref[...]

    h1 = (jnp.dot(s_ref[_PAD - 1:H + _PAD - 1], a_ref[0],
                  preferred_element_type=jnp.float32)
          + jnp.dot(s_ref[_PAD:H + _PAD], a_ref[1],
                    preferred_element_type=jnp.float32)
          + jnp.dot(s_ref[_PAD + 1:H + _PAD + 1], a_ref[2],
                    preferred_element_type=jnp.float32)
          + bias[0:1])
    h1 = jnp.where(h1 > 0, h1, 0.1 * h1)
    h1 = h1.astype(w2_ref.dtype)

    w2b = w2_ref[...]
    for j in range(_P // 2):
        lo = 2 * j * Kout
        h2 = (jnp.dot(h1[:, lo:lo + 2 * Kout], w2b,
                      preferred_element_type=jnp.float32)
              + bias[1:2, lo:lo + 2 * Kout])
        h2 = jnp.maximum(h2, 0.0)
        o_ref[2 * j] = h2[:, 0:Kout]
        o_ref[2 * j + 1] = h2[:, Kout:2 * Kout]


@jax.jit
def _forward(x_nhwc, a, w2, bias):
    N, H, W, Cin = x_nhwc.shape
    Kin = a.shape[1]
    Kout = bias.shape[1]
    Cout = Kout // W

    a_bd = jnp.zeros((3, _P * Kin, _P * Kout), a.dtype)
    for i in range(_P):
        a_bd = a_bd.at[:, i * Kin:(i + 1) * Kin,
                       i * Kout:(i + 1) * Kout].set(a)
    w2_bd = jnp.zeros((2 * Kout, 2 * Kout), w2.dtype)
    for i in range(2):
        w2_bd = w2_bd.at[i * Kout:(i + 1) * Kout,
                         i * Kout:(i + 1) * Kout].set(w2)
    bias_t = jnp.tile(bias, (1, _P))

    x = x_nhwc.reshape(N, H, Kin)

    out = pl.pallas_call(
        _fused_kernel,
        out_shape=jax.ShapeDtypeStruct((N, H, Kout), jnp.float32),
        grid=(N // _P,),
        in_specs=[
            pl.BlockSpec((_P, H, Kin), lambda n: (n, 0, 0)),
            pl.BlockSpec((3, _P * Kin, _P * Kout), lambda n: (0, 0, 0)),
            pl.BlockSpec((2 * Kout, 2 * Kout), lambda n: (0, 0)),
            pl.BlockSpec((2, _P * Kout), lambda n: (0, 0)),
        ],
        out_specs=pl.BlockSpec((_P, H, Kout), lambda n: (n, 0, 0)),
        scratch_shapes=[pltpu.VMEM((H + 2 * _PAD, _P * Kin), a.dtype)],
        compiler_params=pltpu.CompilerParams(
            dimension_semantics=("parallel",),
        ),
    )(x, a_bd, w2_bd, bias_t)

    return out.reshape(N, H, W, Cout)


def kernel(x_nhwc, a, w2, bias):
    return _forward(x_nhwc, a, w2, bias)


# speedup vs baseline: 3.4543x; 2.4527x over previous
import jax
import jax.numpy as jnp
from jax.experimental import pallas as pl
from jax.experimental.pallas import tpu as pltpu

_B = 4


def _fused_kernel(x_ref, a_ref, w2_ref, b_ref, o_ref):
    W, Cin, H = x_ref.shape[1], x_ref.shape[2], x_ref.shape[3]
    Kin = W * Cin

    a0 = a_ref[0]
    a1 = a_ref[1]
    a2 = a_ref[2]
    w2t = w2_ref[...]
    b0 = b_ref[:, 0:1]
    b1 = b_ref[:, 1:2]

    for i in range(_B):
        x = x_ref[i].reshape(Kin, H).astype(a0.dtype)
        z = jnp.zeros((Kin, 1), x.dtype)
        xm = jnp.concatenate([z, x[:, :H - 1]], axis=1)
        xp = jnp.concatenate([x[:, 1:], z], axis=1)

        h1 = (jnp.dot(a0, xm, preferred_element_type=jnp.float32)
              + jnp.dot(a1, x, preferred_element_type=jnp.float32)
              + jnp.dot(a2, xp, preferred_element_type=jnp.float32)
              + b0)
        h1 = jnp.where(h1 > 0, h1, 0.1 * h1)

        h2 = (jnp.dot(w2t, h1.astype(w2t.dtype),
                      preferred_element_type=jnp.float32)
              + b1)
        o_ref[i] = jnp.maximum(h2, 0.0)


@jax.jit
def _forward(x_nhwc, a, w2, bias):
    N, H, W, Cin = x_nhwc.shape
    Kout = bias.shape[1]
    Cout = Kout // W

    x_t = jnp.transpose(x_nhwc, (0, 2, 3, 1))

    a_t = jnp.transpose(a, (0, 2, 1))
    w2_t = w2.T
    bias_t = bias.T

    out = pl.pallas_call(
        _fused_kernel,
        out_shape=jax.ShapeDtypeStruct((N, Kout, H), jnp.float32),
        grid=(N // _B,),
        in_specs=[
            pl.BlockSpec((_B, W, Cin, H), lambda n: (n, 0, 0, 0)),
            pl.BlockSpec((3, Kout, W * Cin), lambda n: (0, 0, 0)),
            pl.BlockSpec((Kout, Kout), lambda n: (0, 0)),
            pl.BlockSpec((Kout, 2), lambda n: (0, 0)),
        ],
        out_specs=pl.BlockSpec((_B, Kout, H), lambda n: (n, 0, 0)),
        compiler_params=pltpu.CompilerParams(
            dimension_semantics=("parallel",),
        ),
    )(x_t, a_t, w2_t, bias_t)

    return out.reshape(N, W, Cout, H).transpose(0, 3, 1, 2)


def kernel(x_nhwc, a, w2, bias):
    return _forward(x_nhwc, a, w2, bias)


# speedup vs baseline: 3.8471x; 1.1137x over previous
import jax
import jax.numpy as jnp
from jax.experimental import pallas as pl
from jax.experimental.pallas import tpu as pltpu

_B = 8


def _fused_kernel(x_ref, a_ref, w2_ref, b_ref, o_ref):
    W, Cin, H = x_ref.shape[1], x_ref.shape[2], x_ref.shape[3]
    Kin = W * Cin

    a0 = a_ref[0]
    a1 = a_ref[1]
    a2 = a_ref[2]
    w2t = w2_ref[...]
    b0 = b_ref[:, 0:1]
    b1 = b_ref[:, 1:2]

    for i in range(_B):
        x = x_ref[i].reshape(Kin, H).astype(a0.dtype)
        z = jnp.zeros((Kin, 1), x.dtype)
        xm = jnp.concatenate([z, x[:, :H - 1]], axis=1)
        xp = jnp.concatenate([x[:, 1:], z], axis=1)

        h1 = (jnp.dot(a0, xm, preferred_element_type=jnp.float32)
              + jnp.dot(a1, x, preferred_element_type=jnp.float32)
              + jnp.dot(a2, xp, preferred_element_type=jnp.float32)
              + b0)
        h1 = jnp.where(h1 > 0, h1, 0.1 * h1)

        h2 = (jnp.dot(w2t, h1.astype(w2t.dtype),
                      preferred_element_type=jnp.float32)
              + b1)
        o_ref[i] = jnp.maximum(h2, 0.0)


@jax.jit
def _forward(x_nhwc, a, w2, bias):
    N, H, W, Cin = x_nhwc.shape
    Kout = bias.shape[1]
    Cout = Kout // W

    x_t = jnp.transpose(x_nhwc, (0, 2, 3, 1))

    a_t = jnp.transpose(a, (0, 2, 1))
    w2_t = w2.T
    bias_t = bias.T

    out = pl.pallas_call(
        _fused_kernel,
        out_shape=jax.ShapeDtypeStruct((N, Kout, H), jnp.float32),
        grid=(N // _B,),
        in_specs=[
            pl.BlockSpec((_B, W, Cin, H), lambda n: (n, 0, 0, 0)),
            pl.BlockSpec((3, Kout, W * Cin), lambda n: (0, 0, 0)),
            pl.BlockSpec((Kout, Kout), lambda n: (0, 0)),
            pl.BlockSpec((Kout, 2), lambda n: (0, 0)),
        ],
        out_specs=pl.BlockSpec((_B, Kout, H), lambda n: (n, 0, 0)),
        compiler_params=pltpu.CompilerParams(
            dimension_semantics=("parallel",),
        ),
    )(x_t, a_t, w2_t, bias_t)

    return out.reshape(N, W, Cout, H).transpose(0, 3, 1, 2)


def kernel(x_nhwc, a, w2, bias):
    return _forward(x_nhwc, a, w2, bias)


# speedup vs baseline: 3.9764x; 1.0336x over previous
import jax
import jax.numpy as jnp
from jax.experimental import pallas as pl
from jax.experimental.pallas import tpu as pltpu

_B = 16


def _fused_kernel(x_ref, a_ref, w2_ref, b_ref, o_ref):
    W, Cin, H = x_ref.shape[1], x_ref.shape[2], x_ref.shape[3]
    Kin = W * Cin

    a0 = a_ref[0]
    a1 = a_ref[1]
    a2 = a_ref[2]
    w2t = w2_ref[...]
    b0 = b_ref[:, 0:1]
    b1 = b_ref[:, 1:2]

    for i in range(_B):
        x = x_ref[i].reshape(Kin, H).astype(a0.dtype)
        z = jnp.zeros((Kin, 1), x.dtype)
        xm = jnp.concatenate([z, x[:, :H - 1]], axis=1)
        xp = jnp.concatenate([x[:, 1:], z], axis=1)

        h1 = (jnp.dot(a0, xm, preferred_element_type=jnp.float32)
              + jnp.dot(a1, x, preferred_element_type=jnp.float32)
              + jnp.dot(a2, xp, preferred_element_type=jnp.float32)
              + b0)
        h1 = jnp.where(h1 > 0, h1, 0.1 * h1)

        h2 = (jnp.dot(w2t, h1.astype(w2t.dtype),
                      preferred_element_type=jnp.float32)
              + b1)
        o_ref[i] = jnp.maximum(h2, 0.0)


@jax.jit
def _forward(x_nhwc, a, w2, bias):
    N, H, W, Cin = x_nhwc.shape
    Kout = bias.shape[1]
    Cout = Kout // W

    x_t = jnp.transpose(x_nhwc, (0, 2, 3, 1))

    a_t = jnp.transpose(a, (0, 2, 1))
    w2_t = w2.T
    bias_t = bias.T

    out = pl.pallas_call(
        _fused_kernel,
        out_shape=jax.ShapeDtypeStruct((N, Kout, H), jnp.float32),
        grid=(N // _B,),
        in_specs=[
            pl.BlockSpec((_B, W, Cin, H), lambda n: (n, 0, 0, 0)),
            pl.BlockSpec((3, Kout, W * Cin), lambda n: (0, 0, 0)),
            pl.BlockSpec((Kout, Kout), lambda n: (0, 0)),
            pl.BlockSpec((Kout, 2), lambda n: (0, 0)),
        ],
        out_specs=pl.BlockSpec((_B, Kout, H), lambda n: (n, 0, 0)),
        compiler_params=pltpu.CompilerParams(
            dimension_semantics=("parallel",),
        ),
    )(x_t, a_t, w2_t, bias_t)

    return out.reshape(N, W, Cout, H).transpose(0, 3, 1, 2)


def kernel(x_nhwc, a, w2, bias):
    return _forward(x_nhwc, a, w2, bias)


# speedup vs baseline: 4.1694x; 1.0486x over previous
import jax
import jax.numpy as jnp
from jax.experimental import pallas as pl
from jax.experimental.pallas import tpu as pltpu

_B = 16


def _fused_kernel(x_ref, a_ref, w2_ref, b_ref, o_ref):
    W, Cin, H = x_ref.shape[1], x_ref.shape[2], x_ref.shape[3]
    Kin = W * Cin

    def wdot(w, v):
        return jax.lax.dot_general(
            w, v, (((0,), (0,)), ((), ())),
            preferred_element_type=jnp.float32)

    a0 = a_ref[0]
    a1 = a_ref[1]
    a2 = a_ref[2]
    w2t = w2_ref[...]
    bcol = b_ref[...].T
    b0 = bcol[:, 0:1]
    b1 = bcol[:, 1:2]

    for i in range(x_ref.shape[0]):
        x = x_ref[i].reshape(Kin, H).astype(a0.dtype)
        z = jnp.zeros((Kin, 1), x.dtype)
        xm = jnp.concatenate([z, x[:, :H - 1]], axis=1)
        xp = jnp.concatenate([x[:, 1:], z], axis=1)

        h1 = wdot(a0, xm) + wdot(a1, x) + wdot(a2, xp) + b0
        h1 = jnp.where(h1 > 0, h1, 0.1 * h1)

        h2 = wdot(w2t, h1.astype(w2t.dtype)) + b1
        o_ref[i] = jnp.maximum(h2, 0.0)


@jax.jit
def _forward(x_nhwc, a, w2, bias):
    N, H, W, Cin = x_nhwc.shape
    Kout = bias.shape[1]
    Cout = Kout // W

    x_t = jnp.transpose(x_nhwc, (0, 2, 3, 1))

    B = min(_B, N)
    out = pl.pallas_call(
        _fused_kernel,
        out_shape=jax.ShapeDtypeStruct((N, Kout, H), jnp.float32),
        grid=(N // B,),
        in_specs=[
            pl.BlockSpec((B, W, Cin, H), lambda n: (n, 0, 0, 0)),
            pl.BlockSpec((3, W * Cin, Kout), lambda n: (0, 0, 0)),
            pl.BlockSpec((Kout, Kout), lambda n: (0, 0)),
            pl.BlockSpec((2, Kout), lambda n: (0, 0)),
        ],
        out_specs=pl.BlockSpec((B, Kout, H), lambda n: (n, 0, 0)),
        compiler_params=pltpu.CompilerParams(
            dimension_semantics=("parallel",),
        ),
    )(x_t, a, w2, bias)

    return out.reshape(N, W, Cout, H).transpose(0, 3, 1, 2)


def kernel(x_nhwc, a, w2, bias):
    return _forward(x_nhwc, a, w2, bias)
